# baseline (speedup 1.0000x reference)
import dataclasses
import functools

import jax
import jax.numpy as jnp
from jax import lax
from jax.experimental import pallas as pl
from jax.experimental.pallas import tpu as pltpu
from jax.experimental.pallas import tpu_sc as plsc

VOCAB_N = 1000000
D = 64
PAD = 0

NC = 2
NS = 16
NW = NC * NS

B, S = 4096, 200
N = B * S
PER_W = N // NW
W = 128
CHUNKS = PER_W // W
RING = 4
G = CHUNKS // RING

LANES = 16
SCALE = 8.0


def _scale_chunk(idx_v, k, gbuf, obuf):
    @pl.loop(0, W // LANES)
    def _(g2):
        r0 = g2 * LANES
        iv = idx_v[k, pl.ds(r0, LANES)]
        pad = iv == PAD
        for j in range(LANES):
            r = r0 + j
            for c in range(0, D, LANES):
                obuf[r, pl.ds(c, LANES)] = gbuf[r, pl.ds(c, LANES)] * SCALE

        @pl.when(jnp.any(pad))
        def _():
            rows = r0 + lax.iota(jnp.int32, LANES)
            zeros = jnp.zeros((LANES,), jnp.float32)
            for c in range(D):
                cols = jnp.full((LANES,), c, jnp.int32)
                plsc.store_scatter(obuf, [rows, cols], zeros, mask=pad)


def _emb_kernel(idx_hbm, table_hbm, out_hbm, idx_v, gbufs, obufs, sem_g, sem_o):
    wid = lax.axis_index("c") * NS + lax.axis_index("s")
    base = wid * PER_W

    pltpu.sync_copy(idx_hbm.at[wid], idx_v)

    for b in range(RING):
        pltpu.make_async_copy(
            table_hbm.at[idx_v.at[b]], gbufs.at[b], sem_g
        ).start()

    @pl.loop(0, G)
    def _(g):
        for b in range(RING):
            k = g * RING + b
            pltpu.make_async_copy(
                table_hbm.at[idx_v.at[k]], gbufs.at[b], sem_g
            ).wait()

            @pl.when(g > 0)
            def _():
                pltpu.make_async_copy(
                    obufs.at[b],
                    out_hbm.at[pl.ds(base + (k - RING) * W, W)],
                    sem_o,
                ).wait()

            _scale_chunk(idx_v, k, gbufs.at[b], obufs.at[b])

            pltpu.make_async_copy(
                obufs.at[b], out_hbm.at[pl.ds(base + k * W, W)], sem_o
            ).start()

            @pl.when(g < G - 1)
            def _():
                pltpu.make_async_copy(
                    table_hbm.at[idx_v.at[k + RING]], gbufs.at[b], sem_g
                ).start()

    for b in range(RING):
        k = (G - 1) * RING + b
        pltpu.make_async_copy(
            obufs.at[b], out_hbm.at[pl.ds(base + k * W, W)], sem_o
        ).wait()


@jax.jit
def _embed(idx3, table):
    mesh = plsc.VectorSubcoreMesh(core_axis_name="c", subcore_axis_name="s")
    cp = pltpu.CompilerParams(
        needs_layout_passes=False, use_tc_tiling_on_sc=False
    )
    run = functools.partial(
        pl.kernel,
        mesh=mesh,
        compiler_params=cp,
        out_type=jax.ShapeDtypeStruct((N, D), jnp.float32),
        scratch_types=[
            pltpu.VMEM((CHUNKS, W), jnp.int32),
            pltpu.VMEM((RING, W, D), jnp.float32),
            pltpu.VMEM((RING, W, D), jnp.float32),
            pltpu.SemaphoreType.DMA,
            pltpu.SemaphoreType.DMA,
        ],
    )(_emb_kernel)
    return run(idx3, table)


def kernel(input, lookup_table):
    idx3 = input.reshape(NW, CHUNKS, W)
    out = _embed(idx3, lookup_table)
    return out.reshape(B, S, D)
